# baseline (speedup 1.0000x reference)
import functools

import jax
import jax.numpy as jnp
from jax import lax
from jax.experimental import pallas as pl
from jax.experimental.pallas import tpu as pltpu
from jax.experimental.pallas import tpu_sc as plsc

NPIX = 100000
NCATS = 50000
NMAP = 1600000
B = 4
T = 8
D = B * T

NC = 2
NS = 16
H = NCATS // NC
ZB = 224
ZR = 7 * ZB
ACC_ROWS = NS * ZR
LASTR = H - (NS - 1) * ZR
STAGE = 1024
STEP = 128
NSTEP = STAGE // STEP
NSLOT = 4
PSTRIPE = NPIX // NS
PCH = 625
NPCH = PSTRIPE // PCH


def _sc_body(run_hbm, src_hbm, dst_hbm, w_hbm, lo_hbm, hi_hbm,
             out_hbm, x_hbm,
             acc, st_src, st_dst, st_w, src_i, rel_i, w_s,
             rows, zbuf, pin, pxc, wb, lo_v, hi_v,
             stsem0, stsem1, gsem0, gsem1, gsem2, gsem3,
             ssem0, ssem1, ssem2, ssem3, psem):
    stsem = (stsem0, stsem1)
    gsem = (gsem0, gsem1, gsem2, gsem3)
    ssem = (ssem0, ssem1, ssem2, ssem3)

    c = lax.axis_index("c")
    s = lax.axis_index("s")
    lane = lax.iota(jnp.int32, 16)
    wid = c * NS + s
    laneq = lane // 8
    laner = lane - laneq * 8

    pltpu.sync_copy(lo_hbm, lo_v)
    pltpu.sync_copy(hi_hbm, hi_v)
    lo = lo_v[pl.ds(wid, 16)][0]
    hi = hi_v[pl.ds(wid, 16)][0]

    p0s = s * PSTRIPE
    for k in range(NPCH):
        p0 = p0s + k * PCH
        for b in range(B):
            pltpu.async_copy(run_hbm.at[b, pl.ds(p0, PCH)], pin.at[b], psem)
        for b in range(B):
            pltpu.make_async_copy(run_hbm.at[b, pl.ds(p0, PCH)], pin.at[b], psem).wait()

        @plsc.parallel_loop(0, PCH, step=1, unroll=4)
        def _(p):
            pv = jnp.full((16,), p, jnp.int32)
            lo16 = plsc.load_gather(pin, [laneq, pv, laner])
            hi16 = plsc.load_gather(pin, [laneq + 2, pv, laner])
            pxc[p, pl.ds(0, 16)] = lo16
            pxc[p, pl.ds(16, 16)] = hi16

        pltpu.sync_copy(pxc, x_hbm.at[pl.ds(p0, PCH)])

    zf = jnp.zeros((16,), jnp.float32)

    def zrow(r, _):
        zbuf[r, pl.ds(0, 16)] = zf
        zbuf[r, pl.ds(16, 16)] = zf
        return 0

    lax.fori_loop(0, ZB, zrow, 0)
    for k in range(ZR // ZB):
        pltpu.sync_copy(zbuf, acc.at[pl.ds(s * ZR + k * ZB, ZB)])
    plsc.subcore_barrier()

    base0 = (lo // 8) * 8
    nblk = (hi - base0 + STAGE - 1) // STAGE
    dbase = c * H

    def stage_gsd(kb):
        return jnp.minimum(base0 + kb * STAGE, NMAP - STAGE)

    def fire_stage(kb, par):
        gsd = stage_gsd(kb)
        pltpu.async_copy(src_hbm.at[pl.ds(gsd, STAGE)], st_src.at[par], stsem[par])
        pltpu.async_copy(dst_hbm.at[pl.ds(gsd, STAGE)], st_dst.at[par], stsem[par])
        pltpu.async_copy(w_hbm.at[pl.ds(gsd, STAGE)], st_w.at[par], stsem[par])

    def wait_stage(kb, par):
        gsd = stage_gsd(kb)
        pltpu.make_async_copy(src_hbm.at[pl.ds(gsd, STAGE)], st_src.at[par], stsem[par]).wait()
        pltpu.make_async_copy(dst_hbm.at[pl.ds(gsd, STAGE)], st_dst.at[par], stsem[par]).wait()
        pltpu.make_async_copy(w_hbm.at[pl.ds(gsd, STAGE)], st_w.at[par], stsem[par]).wait()

    def build(par, gsd, lo_eff, i, sl):
        off = i * STEP
        for p in range(STEP // 16):
            q = off + p * 16
            gidx = gsd + q + lane
            m = jnp.logical_and(gidx >= lo_eff, gidx < hi)
            sp = st_src[par, pl.ds(q, 16)]
            dp = st_dst[par, pl.ds(q, 16)]
            wp = st_w[par, pl.ds(q, 16)]
            rel = jnp.clip(dp - dbase, 0, ACC_ROWS - 1)
            src_i[sl, pl.ds(p * 16, 16)] = sp
            rel_i[sl, pl.ds(p * 16, 16)] = rel
            w_s[sl, pl.ds(p * 16, 16)] = jnp.where(m, wp, 0.0)

    def fire_gather(sl):
        pltpu.async_copy(x_hbm.at[src_i.at[sl]], rows.at[sl], gsem[sl])

    def wait_gather(sl):
        pltpu.make_async_copy(x_hbm.at[src_i.at[sl]], rows.at[sl], gsem[sl]).wait()

    def fire_scatter(sl):
        pltpu.async_copy(rows.at[sl], acc.at[rel_i.at[sl]], ssem[sl], add=True)

    def wait_scatter(sl):
        pltpu.make_async_copy(rows.at[sl], acc.at[rel_i.at[sl]], ssem[sl]).wait()

    def multiply(sl):
        @plsc.parallel_loop(0, STEP, step=16, unroll=2)
        def _(j0):
            wv = w_s[sl, pl.ds(j0, 16)]
            for l in range(16):
                wj = jnp.full((16,), wv[l])
                rows[sl, j0 + l, pl.ds(0, 16)] = rows[sl, j0 + l, pl.ds(0, 16)] * wj
                rows[sl, j0 + l, pl.ds(16, 16)] = rows[sl, j0 + l, pl.ds(16, 16)] * wj

    @pl.when(nblk >= 1)
    def _():
        fire_stage(0, 0)

    @pl.when(nblk >= 2)
    def _():
        fire_stage(1, 1)

    nblk2 = (nblk + 1) // 2

    def blk2_body(kb2, _):
        for par in range(2):
            kb = kb2 * 2 + par

            @pl.when(kb < nblk)
            def _():
                gsd = stage_gsd(kb)
                gs = base0 + kb * STAGE
                lo_eff = jnp.maximum(lo, gs)
                wait_stage(kb, par)
                build(par, gsd, lo_eff, 0, 0)
                fire_gather(0)
                for i in range(NSTEP):
                    sl = i % NSLOT
                    if i + 1 < NSTEP:
                        nsl = (i + 1) % NSLOT
                        if i >= 3:
                            wait_scatter(nsl)
                        build(par, gsd, lo_eff, i + 1, nsl)
                        fire_gather(nsl)
                    wait_gather(sl)
                    multiply(sl)
                    fire_scatter(sl)
                for sl in range(NSLOT):
                    wait_scatter(sl)
                @pl.when(kb + 2 < nblk)
                def _():
                    fire_stage(kb + 2, par)

        return 0

    lax.fori_loop(0, nblk2, blk2_body, 0)
    plsc.subcore_barrier()

    obase = dbase + s * ZR

    def wb_extract(k):
        pltpu.sync_copy(acc.at[pl.ds(s * ZR + k * ZB, ZB)], zbuf)

        @plsc.parallel_loop(0, ZB // 2, step=1, unroll=4)
        def _(pr):
            rowv = jnp.full((16,), pr * 2, jnp.int32) + laneq
            for b in range(B):
                vals = plsc.load_gather(zbuf, [rowv, laner + b * T])
                plsc.store_scatter(wb.at[b], [rowv, laner], vals)

    def wb_full(k):
        wb_extract(k)
        for b in range(B):
            pltpu.sync_copy(wb.at[b], out_hbm.at[b, pl.ds(obase + k * ZB, ZB)])

    def wb_partial(k, nrows):
        wb_extract(k)
        for b in range(B):
            pltpu.sync_copy(wb.at[b, pl.ds(0, nrows)],
                            out_hbm.at[b, pl.ds(obase + k * ZB, nrows)])

    for k in range(ZR // ZB - 1):
        wb_full(k)

    @pl.when(s < NS - 1)
    def _():
        wb_full(ZR // ZB - 1)

    @pl.when(s == NS - 1)
    def _():
        wb_partial(ZR // ZB - 1, LASTR - (ZR // ZB - 1) * ZB)


_interp_sc = functools.partial(
    pl.kernel,
    out_type=(
        jax.ShapeDtypeStruct((B, NCATS, T), jnp.float32),
        jax.ShapeDtypeStruct((NPIX, D), jnp.float32),
    ),
    mesh=plsc.VectorSubcoreMesh(core_axis_name="c", subcore_axis_name="s"),
    compiler_params=pltpu.CompilerParams(
        needs_layout_passes=False, use_tc_tiling_on_sc=False),
    scratch_types=[
        pltpu.VMEM_SHARED((ACC_ROWS, D), jnp.float32),
        pltpu.VMEM((2, STAGE), jnp.int32),
        pltpu.VMEM((2, STAGE), jnp.int32),
        pltpu.VMEM((2, STAGE), jnp.float32),
        pltpu.VMEM((NSLOT, STEP), jnp.int32),
        pltpu.VMEM((NSLOT, STEP), jnp.int32),
        pltpu.VMEM((NSLOT, STEP), jnp.float32),
        pltpu.VMEM((NSLOT, STEP, D), jnp.float32),
        pltpu.VMEM((ZB, D), jnp.float32),
        pltpu.VMEM((B, PCH, T), jnp.float32),
        pltpu.VMEM((PCH, D), jnp.float32),
        pltpu.VMEM((B, ZB, T), jnp.float32),
        pltpu.VMEM((48,), jnp.int32),
        pltpu.VMEM((48,), jnp.int32),
        pltpu.SemaphoreType.DMA,
        pltpu.SemaphoreType.DMA,
        pltpu.SemaphoreType.DMA,
        pltpu.SemaphoreType.DMA,
        pltpu.SemaphoreType.DMA,
        pltpu.SemaphoreType.DMA,
        pltpu.SemaphoreType.DMA,
        pltpu.SemaphoreType.DMA,
        pltpu.SemaphoreType.DMA,
        pltpu.SemaphoreType.DMA,
        pltpu.SemaphoreType.DMA,
    ],
)(_sc_body)


def kernel(runoff, src_idxs, dest_idxs, weights):
    split = jnp.searchsorted(dest_idxs, H).astype(jnp.int32)
    s17 = jnp.arange(NS + 1, dtype=jnp.int32)
    b0 = (s17 * split) // NS
    b1 = split + (s17 * (NMAP - split)) // NS
    pad = jnp.zeros((NS,), jnp.int32)
    lo_arr = jnp.concatenate([b0[:NS], b1[:NS], pad])
    hi_arr = jnp.concatenate([b0[1:], b1[1:], pad])
    out, _ = _interp_sc(runoff, src_idxs.astype(jnp.int32),
                        dest_idxs.astype(jnp.int32), weights, lo_arr, hi_arr)
    return out


# speedup vs baseline: 1.4857x; 1.4857x over previous
import functools

import jax
import jax.numpy as jnp
from jax import lax
from jax.experimental import pallas as pl
from jax.experimental.pallas import tpu as pltpu
from jax.experimental.pallas import tpu_sc as plsc

NPIX = 100000
NCATS = 50000
NMAP = 1600000
B = 4
T = 8
D = B * T

NC = 2
NS = 16
H = NCATS // NC
ZB = 224
ZR = 7 * ZB
ACC_ROWS = NS * ZR
STAGE = 1024
STEP = 128
NSTEP = STAGE // STEP
NSLOT = 4


def _sc_body(x_hbm, src_hbm, dst_hbm, w_hbm, lo_hbm, hi_hbm, out_hbm,
             acc, st_src, st_dst, st_w, src_i, rel_i, w_s,
             rows, zbuf, lo_v, hi_v,
             stsem0, stsem1, gsem0, gsem1, gsem2, gsem3,
             ssem0, ssem1, ssem2, ssem3):
    stsem = (stsem0, stsem1)
    gsem = (gsem0, gsem1, gsem2, gsem3)
    ssem = (ssem0, ssem1, ssem2, ssem3)

    c = lax.axis_index("c")
    s = lax.axis_index("s")
    lane = lax.iota(jnp.int32, 16)
    wid = c * NS + s

    pltpu.sync_copy(lo_hbm, lo_v)
    pltpu.sync_copy(hi_hbm, hi_v)
    lo = lo_v[pl.ds(wid, 16)][0]
    hi = hi_v[pl.ds(wid, 16)][0]

    zf = jnp.zeros((16,), jnp.float32)

    def zrow(r, _):
        zbuf[r, pl.ds(0, 16)] = zf
        zbuf[r, pl.ds(16, 16)] = zf
        return 0

    lax.fori_loop(0, ZB, zrow, 0)
    for k in range(ZR // ZB):
        pltpu.sync_copy(zbuf, acc.at[pl.ds(s * ZR + k * ZB, ZB)])
    plsc.subcore_barrier()

    base0 = (lo // 8) * 8
    nblk = (hi - base0 + STAGE - 1) // STAGE
    dbase = c * H

    def stage_gsd(kb):
        return jnp.minimum(base0 + kb * STAGE, NMAP - STAGE)

    def fire_stage(kb, par):
        gsd = stage_gsd(kb)
        pltpu.async_copy(src_hbm.at[pl.ds(gsd, STAGE)], st_src.at[par], stsem[par])
        pltpu.async_copy(dst_hbm.at[pl.ds(gsd, STAGE)], st_dst.at[par], stsem[par])
        pltpu.async_copy(w_hbm.at[pl.ds(gsd, STAGE)], st_w.at[par], stsem[par])

    def wait_stage(kb, par):
        gsd = stage_gsd(kb)
        pltpu.make_async_copy(src_hbm.at[pl.ds(gsd, STAGE)], st_src.at[par], stsem[par]).wait()
        pltpu.make_async_copy(dst_hbm.at[pl.ds(gsd, STAGE)], st_dst.at[par], stsem[par]).wait()
        pltpu.make_async_copy(w_hbm.at[pl.ds(gsd, STAGE)], st_w.at[par], stsem[par]).wait()

    def build(par, gsd, lo_eff, i, sl):
        off = i * STEP
        for p in range(STEP // 16):
            q = off + p * 16
            gidx = gsd + q + lane
            m = jnp.logical_and(gidx >= lo_eff, gidx < hi)
            sp = st_src[par, pl.ds(q, 16)]
            dp = st_dst[par, pl.ds(q, 16)]
            wp = st_w[par, pl.ds(q, 16)]
            rel = jnp.clip(dp - dbase, 0, ACC_ROWS - 1)
            src_i[sl, pl.ds(p * 16, 16)] = sp
            rel_i[sl, pl.ds(p * 16, 16)] = rel
            w_s[sl, pl.ds(p * 16, 16)] = jnp.where(m, wp, 0.0)

    def fire_gather(sl):
        pltpu.async_copy(x_hbm.at[src_i.at[sl]], rows.at[sl], gsem[sl])

    def wait_gather(sl):
        pltpu.make_async_copy(x_hbm.at[src_i.at[sl]], rows.at[sl], gsem[sl]).wait()

    def fire_scatter(sl):
        pltpu.async_copy(rows.at[sl], acc.at[rel_i.at[sl]], ssem[sl], add=True)

    def wait_scatter(sl):
        pltpu.make_async_copy(rows.at[sl], acc.at[rel_i.at[sl]], ssem[sl]).wait()

    def multiply(sl):
        @plsc.parallel_loop(0, STEP, step=16, unroll=2)
        def _(j0):
            wv = w_s[sl, pl.ds(j0, 16)]
            for l in range(16):
                wj = jnp.full((16,), wv[l])
                rows[sl, j0 + l, pl.ds(0, 16)] = rows[sl, j0 + l, pl.ds(0, 16)] * wj
                rows[sl, j0 + l, pl.ds(16, 16)] = rows[sl, j0 + l, pl.ds(16, 16)] * wj

    @pl.when(nblk >= 1)
    def _():
        fire_stage(0, 0)

    @pl.when(nblk >= 2)
    def _():
        fire_stage(1, 1)

    nblk2 = (nblk + 1) // 2

    def blk2_body(kb2, _):
        for par in range(2):
            kb = kb2 * 2 + par

            @pl.when(kb < nblk)
            def _():
                gsd = stage_gsd(kb)
                gs = base0 + kb * STAGE
                lo_eff = jnp.maximum(lo, gs)
                wait_stage(kb, par)

                def deferred_wait(slot):
                    if par == 0:
                        @pl.when(kb > 0)
                        def _():
                            wait_scatter(slot)
                    else:
                        wait_scatter(slot)

                deferred_wait(0)
                build(par, gsd, lo_eff, 0, 0)
                fire_gather(0)
                for i in range(NSTEP):
                    sl = i % NSLOT
                    if i + 1 < NSTEP:
                        nsl = (i + 1) % NSLOT
                        if i >= 3:
                            wait_scatter(nsl)
                        else:
                            deferred_wait(nsl)
                        build(par, gsd, lo_eff, i + 1, nsl)
                        fire_gather(nsl)
                    wait_gather(sl)
                    multiply(sl)
                    fire_scatter(sl)
                @pl.when(kb + 2 < nblk)
                def _():
                    fire_stage(kb + 2, par)

        return 0

    lax.fori_loop(0, nblk2, blk2_body, 0)

    @pl.when(nblk >= 1)
    def _():
        for sl in range(NSLOT):
            wait_scatter(sl)

    plsc.subcore_barrier()

    outbase = c * ACC_ROWS + s * ZR
    for k in range(ZR // ZB):
        pltpu.sync_copy(acc.at[pl.ds(s * ZR + k * ZB, ZB)], zbuf)
        pltpu.sync_copy(zbuf, out_hbm.at[pl.ds(outbase + k * ZB, ZB)])


_interp_sc = functools.partial(
    pl.kernel,
    out_type=jax.ShapeDtypeStruct((NC * ACC_ROWS, D), jnp.float32),
    mesh=plsc.VectorSubcoreMesh(core_axis_name="c", subcore_axis_name="s"),
    compiler_params=pltpu.CompilerParams(
        needs_layout_passes=False, use_tc_tiling_on_sc=False),
    scratch_types=[
        pltpu.VMEM_SHARED((ACC_ROWS, D), jnp.float32),
        pltpu.VMEM((2, STAGE), jnp.int32),
        pltpu.VMEM((2, STAGE), jnp.int32),
        pltpu.VMEM((2, STAGE), jnp.float32),
        pltpu.VMEM((NSLOT, STEP), jnp.int32),
        pltpu.VMEM((NSLOT, STEP), jnp.int32),
        pltpu.VMEM((NSLOT, STEP), jnp.float32),
        pltpu.VMEM((NSLOT, STEP, D), jnp.float32),
        pltpu.VMEM((ZB, D), jnp.float32),
        pltpu.VMEM((48,), jnp.int32),
        pltpu.VMEM((48,), jnp.int32),
        pltpu.SemaphoreType.DMA,
        pltpu.SemaphoreType.DMA,
        pltpu.SemaphoreType.DMA,
        pltpu.SemaphoreType.DMA,
        pltpu.SemaphoreType.DMA,
        pltpu.SemaphoreType.DMA,
        pltpu.SemaphoreType.DMA,
        pltpu.SemaphoreType.DMA,
        pltpu.SemaphoreType.DMA,
        pltpu.SemaphoreType.DMA,
    ],
)(_sc_body)


def kernel(runoff, src_idxs, dest_idxs, weights):
    x = runoff.transpose(1, 0, 2).reshape(NPIX, D)
    split = jnp.searchsorted(dest_idxs, H).astype(jnp.int32)
    s17 = jnp.arange(NS + 1, dtype=jnp.int32)
    b0 = (s17 * split) // NS
    b1 = split + (s17 * (NMAP - split)) // NS
    pad = jnp.zeros((NS,), jnp.int32)
    lo_arr = jnp.concatenate([b0[:NS], b1[:NS], pad])
    hi_arr = jnp.concatenate([b0[1:], b1[1:], pad])
    outk = _interp_sc(x, src_idxs.astype(jnp.int32), dest_idxs.astype(jnp.int32),
                      weights, lo_arr, hi_arr)
    out = jnp.concatenate([outk[:H], outk[ACC_ROWS:ACC_ROWS + H]], axis=0)
    return out.reshape(NCATS, B, T).transpose(1, 0, 2)
